# baseline (speedup 1.0000x reference)
import jax
import jax.numpy as jnp
from jax import lax
from jax.experimental import pallas as pl
from jax.experimental.pallas import tpu as pltpu
from jax.experimental.pallas import tpu_sc as plsc

B, L, D = 4096, 200, 64
NC, NS = 2, 16
NW = NC * NS
BT = B // NW
NG = 3
NSO = 2
AHEAD = 2


def _emb_body(idx_hbm, tbl_hbm, posb_hbm, out_hbm, idx_v, posl_v, ridx_v,
              rows_v, stage_v, gsem, psem, osem):
    wid = lax.axis_index("s") * NC + lax.axis_index("c")
    b0 = wid * BT
    pltpu.sync_copy(idx_hbm.at[:, pl.ds(b0, BT)], idx_v)

    items = [lax.iota(jnp.int32, 16) + 16 * g for g in range(BT // 16)]

    def fire(l, sg):
        for g in range(BT // 16):
            sl = pl.ds(16 * g, 16)
            ridx_v[sg, sl] = lax.shift_right_logical(idx_v[l, sl], 1)
        pltpu.async_copy(tbl_hbm.at[ridx_v.at[sg]], rows_v.at[sg],
                         gsem.at[sg])
        pltpu.async_copy(posb_hbm.at[l], posl_v.at[sg], psem.at[sg])

    def wait_gather(l, sg):
        pltpu.make_async_copy(tbl_hbm.at[ridx_v.at[sg]], rows_v.at[sg],
                              gsem.at[sg]).wait()
        pltpu.make_async_copy(posb_hbm.at[l], posl_v.at[sg],
                              psem.at[sg]).wait()

    def wait_out(l, so):
        pltpu.make_async_copy(stage_v.at[so], out_hbm.at[l, :, pl.ds(b0, BT)],
                              osem.at[so]).wait()

    for l in range(AHEAD):
        fire(l, l % NG)

    def pos_body(l, carry):
        ln = l + AHEAD

        @pl.when(ln < L)
        def _():
            fire(ln, lax.rem(ln, NG))

        sg = lax.rem(l, NG)
        so = lax.rem(l, NSO)
        wait_gather(l, sg)

        @pl.when(l >= NSO)
        def _():
            wait_out(l - NSO, so)

        par = [(idx_v[l, pl.ds(16 * g, 16)] & 1) * 64
               for g in range(BT // 16)]
        rows2 = rows_v.at[sg]

        def d_body(dd):
            d = lax.shift_right_logical(dd, 4)
            pos_vec = posl_v[sg, pl.ds(dd, 16)]
            for g in range(BT // 16):
                vals = plsc.load_gather(rows2, [items[g], par[g] + d])
                stage_v[so, d, pl.ds(16 * g, 16)] = vals + pos_vec

        plsc.parallel_loop(0, D * 16, 16, unroll=4)(d_body)
        pltpu.async_copy(stage_v.at[so], out_hbm.at[l, :, pl.ds(b0, BT)],
                         osem.at[so])
        return carry

    lax.fori_loop(0, L, pos_body, 0)

    for k in range(NSO):
        l = L - NSO + k
        wait_out(l, l % NSO)


def kernel(embedding_idx, token_table, pos_table):
    idx_t = embedding_idx.astype(jnp.int32).T
    tbl2 = token_table.reshape(500000, 128)
    posb = jnp.broadcast_to(pos_table[:, :, None],
                            (L, D, 16)).reshape(L, D * 16)
    mesh = plsc.VectorSubcoreMesh(core_axis_name="c", subcore_axis_name="s")
    k = pl.kernel(
        _emb_body,
        out_type=jax.ShapeDtypeStruct((L, D, B), jnp.float32),
        mesh=mesh,
        scratch_types=[
            pltpu.VMEM((L, BT), jnp.int32),
            pltpu.VMEM((NG, D * 16), jnp.float32),
            pltpu.VMEM((NG, BT), jnp.int32),
            pltpu.VMEM((NG, BT, 128), jnp.float32),
            pltpu.VMEM((NSO, D, BT), jnp.float32),
            pltpu.SemaphoreType.DMA((NG,)),
            pltpu.SemaphoreType.DMA((NG,)),
            pltpu.SemaphoreType.DMA((NSO,)),
        ],
        compiler_params=pltpu.CompilerParams(needs_layout_passes=False),
    )
    out_t = k(idx_t, tbl2, posb)
    return out_t.transpose(2, 0, 1)


# speedup vs baseline: 1.4990x; 1.4990x over previous
import jax
import jax.numpy as jnp
import numpy as np
from jax import lax
from jax.experimental import pallas as pl
from jax.experimental.pallas import tpu as pltpu
from jax.experimental.pallas import tpu_sc as plsc

B, L, D = 4096, 200, 64
NC, NS = 2, 16
NW = NC * NS
BT = B // NW
NG = 3
NSO = 2
AHEAD = 2


def _emb_body(idx_hbm, tbl_hbm, posb_hbm, out_hbm, idx_v, posl_v, ridx_v,
              rows_v, stage_v, gsem, psem, osem):
    wid = lax.axis_index("s") * NC + lax.axis_index("c")
    b0 = wid * BT
    pltpu.sync_copy(idx_hbm.at[:, pl.ds(b0, BT)], idx_v)

    iota = lax.iota(jnp.int32, 16)
    items = [iota + 16 * g for g in range(BT // 16)]

    def fire(l, sg):
        for g in range(BT // 16):
            sl = pl.ds(16 * g, 16)
            ridx_v[sg, sl] = lax.shift_right_logical(idx_v[l, sl], 1)
        pltpu.async_copy(tbl_hbm.at[ridx_v.at[sg]], rows_v.at[sg],
                         gsem.at[sg])
        pltpu.async_copy(posb_hbm.at[l], posl_v.at[sg], psem.at[sg])

    def wait_gather(l, sg):
        pltpu.make_async_copy(tbl_hbm.at[ridx_v.at[sg]], rows_v.at[sg],
                              gsem.at[sg]).wait()
        pltpu.make_async_copy(posb_hbm.at[l], posl_v.at[sg],
                              psem.at[sg]).wait()

    def wait_out(l, so):
        pltpu.make_async_copy(stage_v.at[so], out_hbm.at[l, :, pl.ds(b0, BT)],
                              osem.at[so]).wait()

    for l in range(AHEAD):
        fire(l, l % NG)

    def pos_body(l, carry):
        ln = l + AHEAD

        @pl.when(ln < L)
        def _():
            fire(ln, lax.rem(ln, NG))

        sg = lax.rem(l, NG)
        so = lax.rem(l, NSO)
        wait_gather(l, sg)

        @pl.when(l >= NSO)
        def _():
            wait_out(l - NSO, so)

        par = [(idx_v[l, pl.ds(16 * g, 16)] & 1) * 64
               for g in range(BT // 16)]
        rows2 = rows_v.at[sg]

        def d0_body(d0):
            rot = (iota + d0) & 15
            for qd in range(D // 16):
                rotq = rot + 16 * qd
                pos_vec = posl_v[sg, pl.ds(256 * qd + d0 * 16, 16)]
                for g in range(BT // 16):
                    col = par[g] + rotq
                    vals = plsc.load_gather(rows2, [items[g], col])
                    plsc.store_scatter(stage_v.at[so], [rotq, items[g]],
                                       vals + pos_vec)

        plsc.parallel_loop(0, 16, 1, unroll=2)(d0_body)
        pltpu.async_copy(stage_v.at[so], out_hbm.at[l, :, pl.ds(b0, BT)],
                         osem.at[so])
        return carry

    lax.fori_loop(0, L, pos_body, 0)

    for k in range(NSO):
        l = L - NSO + k
        wait_out(l, l % NSO)


def _pos_diag_order():
    i = np.arange(16)
    d0 = np.arange(16)
    qd = np.arange(4)
    dmat = (d0[:, None] + i[None, :]) % 16
    full = qd[:, None, None] * 16 + dmat[None]
    return jnp.asarray(full.reshape(-1), dtype=jnp.int32)


def kernel(embedding_idx, token_table, pos_table):
    idx_t = embedding_idx.astype(jnp.int32).T
    tbl2 = token_table.reshape(500000, 128)
    posb = jnp.take(pos_table, _pos_diag_order(), axis=1)
    mesh = plsc.VectorSubcoreMesh(core_axis_name="c", subcore_axis_name="s")
    k = pl.kernel(
        _emb_body,
        out_type=jax.ShapeDtypeStruct((L, D, B), jnp.float32),
        mesh=mesh,
        scratch_types=[
            pltpu.VMEM((L, BT), jnp.int32),
            pltpu.VMEM((NG, D * 16), jnp.float32),
            pltpu.VMEM((NG, BT), jnp.int32),
            pltpu.VMEM((NG, BT, 128), jnp.float32),
            pltpu.VMEM((NSO, D, BT), jnp.float32),
            pltpu.SemaphoreType.DMA((NG,)),
            pltpu.SemaphoreType.DMA((NG,)),
            pltpu.SemaphoreType.DMA((NSO,)),
        ],
        compiler_params=pltpu.CompilerParams(needs_layout_passes=False),
    )
    out_t = k(idx_t, tbl2, posb)
    return out_t.transpose(2, 0, 1)


# speedup vs baseline: 1.6134x; 1.0763x over previous
import jax
import jax.numpy as jnp
import numpy as np
from jax import lax
from jax.experimental import pallas as pl
from jax.experimental.pallas import tpu as pltpu
from jax.experimental.pallas import tpu_sc as plsc

B, L, D = 4096, 200, 64
NC, NS = 2, 16
NW = NC * NS
BT = B // NW
NG = 3
NSO = 2
AHEAD = 2


def _emb_body(idx_hbm, tbl_hbm, posb_hbm, out_hbm, idx_v, posl_v,
              rows_v, stage_v, gsem, psem, osem):
    wid = lax.axis_index("s") * NC + lax.axis_index("c")
    b0 = wid * BT
    pltpu.sync_copy(idx_hbm.at[:, pl.ds(b0, BT)], idx_v)

    iota = lax.iota(jnp.int32, 16)
    items = [iota + 16 * g for g in range(BT // 16)]

    def fire(l, sg):
        pltpu.async_copy(tbl_hbm.at[idx_v.at[l]], rows_v.at[sg],
                         gsem.at[sg])
        pltpu.async_copy(posb_hbm.at[l], posl_v.at[sg], psem.at[sg])

    def wait_gather(l, sg):
        pltpu.make_async_copy(tbl_hbm.at[idx_v.at[l]], rows_v.at[sg],
                              gsem.at[sg]).wait()
        pltpu.make_async_copy(posb_hbm.at[l], posl_v.at[sg],
                              psem.at[sg]).wait()

    def wait_out(l, so):
        pltpu.make_async_copy(stage_v.at[so], out_hbm.at[l, :, pl.ds(b0, BT)],
                              osem.at[so]).wait()

    for l in range(AHEAD):
        fire(l, l % NG)

    def pos_body(l, carry):
        ln = l + AHEAD

        @pl.when(ln < L)
        def _():
            fire(ln, lax.rem(ln, NG))

        sg = lax.rem(l, NG)
        so = lax.rem(l, NSO)
        wait_gather(l, sg)

        @pl.when(l >= NSO)
        def _():
            wait_out(l - NSO, so)

        rows2 = rows_v.at[sg]

        def d0_body(d0):
            rot = (iota + d0) & 15
            for qd in range(D // 16):
                rotq = rot + 16 * qd
                pos_vec = posl_v[sg, pl.ds(256 * qd + d0 * 16, 16)]
                for g in range(BT // 16):
                    vals = plsc.load_gather(rows2, [items[g], rotq])
                    plsc.store_scatter(stage_v.at[so], [rotq, items[g]],
                                       vals + pos_vec)

        plsc.parallel_loop(0, 16, 1, unroll=2)(d0_body)
        pltpu.async_copy(stage_v.at[so], out_hbm.at[l, :, pl.ds(b0, BT)],
                         osem.at[so])
        return carry

    lax.fori_loop(0, L, pos_body, 0)

    for k in range(NSO):
        l = L - NSO + k
        wait_out(l, l % NSO)


def _pos_diag_order():
    i = np.arange(16)
    d0 = np.arange(16)
    qd = np.arange(4)
    dmat = (d0[:, None] + i[None, :]) % 16
    full = qd[:, None, None] * 16 + dmat[None]
    return jnp.asarray(full.reshape(-1), dtype=jnp.int32)


def kernel(embedding_idx, token_table, pos_table):
    idx_t = embedding_idx.astype(jnp.int32).T
    tbl2 = jnp.pad(token_table, ((0, 0), (0, 64)))
    posb = jnp.take(pos_table, _pos_diag_order(), axis=1)
    mesh = plsc.VectorSubcoreMesh(core_axis_name="c", subcore_axis_name="s")
    k = pl.kernel(
        _emb_body,
        out_type=jax.ShapeDtypeStruct((L, D, B), jnp.float32),
        mesh=mesh,
        scratch_types=[
            pltpu.VMEM((L, BT), jnp.int32),
            pltpu.VMEM((NG, D * 16), jnp.float32),
            pltpu.VMEM((NG, BT, 128), jnp.float32),
            pltpu.VMEM((NSO, D, BT), jnp.float32),
            pltpu.SemaphoreType.DMA((NG,)),
            pltpu.SemaphoreType.DMA((NG,)),
            pltpu.SemaphoreType.DMA((NSO,)),
        ],
        compiler_params=pltpu.CompilerParams(needs_layout_passes=False),
    )
    out_t = k(idx_t, tbl2, posb)
    return out_t.transpose(2, 0, 1)


# speedup vs baseline: 2.3478x; 1.4552x over previous
import jax
import jax.numpy as jnp
import numpy as np
from jax import lax
from jax.experimental import pallas as pl
from jax.experimental.pallas import tpu as pltpu
from jax.experimental.pallas import tpu_sc as plsc

B, L, D = 4096, 200, 64
NC, NS = 2, 16
NW = NC * NS
BT = B // NW
NG = 3
NSO = 2
AHEAD = 2


def _emb_body(idx_hbm, tbl_hbm, posb_hbm, out_hbm, idx_v, posl_v,
              rows_v, stage_v, gsem, psem, osem):
    wid = lax.axis_index("s") * NC + lax.axis_index("c")
    b0 = wid * BT
    pltpu.sync_copy(idx_hbm.at[:, pl.ds(b0, BT)], idx_v)

    iota = lax.iota(jnp.int32, 16)
    items = [iota + 16 * g for g in range(BT // 16)]

    def fire(l, sg):
        pltpu.async_copy(tbl_hbm.at[idx_v.at[l]], rows_v.at[sg],
                         gsem.at[sg])
        pltpu.async_copy(posb_hbm.at[l], posl_v.at[sg], psem.at[sg])

    def wait_gather(l, sg):
        pltpu.make_async_copy(tbl_hbm.at[idx_v.at[l]], rows_v.at[sg],
                              gsem.at[sg]).wait()
        pltpu.make_async_copy(posb_hbm.at[l], posl_v.at[sg],
                              psem.at[sg]).wait()

    def wait_out(l, so):
        pltpu.make_async_copy(stage_v.at[so], out_hbm.at[l, :, pl.ds(b0, BT)],
                              osem.at[so]).wait()

    for l in range(AHEAD):
        fire(l, l % NG)

    def pos_body(l, carry):
        ln = l + AHEAD

        @pl.when(ln < L)
        def _():
            fire(ln, lax.rem(ln, NG))

        sg = lax.rem(l, NG)
        so = lax.rem(l, NSO)
        wait_gather(l, sg)

        @pl.when(l >= NSO)
        def _():
            wait_out(l - NSO, so)

        rows2 = rows_v.at[sg]

        def d0_body(d0):
            rot = (iota + d0) & 15
            for qd in range(D // 16):
                rotq = rot + 16 * qd
                pos_vec = posl_v[sg, pl.ds(256 * qd + d0 * 16, 16)]
                for g in range(BT // 16):
                    vals = plsc.load_gather(rows2, [items[g], rotq])
                    plsc.store_scatter(stage_v.at[so], [rotq, items[g]],
                                       vals + pos_vec)

        plsc.parallel_loop(0, 16, 1, unroll=2)(d0_body)
        pltpu.async_copy(stage_v.at[so], out_hbm.at[l, :, pl.ds(b0, BT)],
                         osem.at[so])
        return carry

    lax.fori_loop(0, L, pos_body, 0)

    for k in range(NSO):
        l = L - NSO + k
        wait_out(l, l % NSO)


VOCAB = 1000000
VP = 1000064
NFULL = 7808
PER = NFULL // NW


def _fmt_body(tblt_hbm, tail_hbm, out_hbm, tin_v, tout_v, isem, fsem):
    wid = lax.axis_index("s") * NC + lax.axis_index("c")
    iota = lax.iota(jnp.int32, 16)
    items = [iota + 16 * g for g in range(8)]

    def fire_in(j, s):
        pltpu.async_copy(tblt_hbm.at[:, pl.ds(j * 128, 128)], tin_v.at[s],
                         isem.at[s])

    def wait_in(j, s):
        pltpu.make_async_copy(tblt_hbm.at[:, pl.ds(j * 128, 128)],
                              tin_v.at[s], isem.at[s]).wait()

    def wait_out(j, s):
        pltpu.make_async_copy(tout_v.at[s], out_hbm.at[pl.ds(j * 128, 128)],
                              fsem.at[s]).wait()

    def transpose(si, so):
        def d0_body(d0):
            rot = (iota + d0) & 15
            for qd in range(D // 16):
                rotq = rot + 16 * qd
                for g in range(8):
                    vals = plsc.load_gather(tin_v.at[si], [rotq, items[g]])
                    plsc.store_scatter(tout_v.at[so], [items[g], rotq], vals)

        plsc.parallel_loop(0, 16, 1, unroll=2)(d0_body)

    j0 = wid * PER
    for i in range(2):
        fire_in(j0 + i, i % 3)

    def blk_body(i, carry):
        j = j0 + i

        @pl.when(i + 2 < PER)
        def _():
            fire_in(j + 2, lax.rem(i + 2, 3))

        si = lax.rem(i, 3)
        so = lax.rem(i, 2)
        wait_in(j, si)

        @pl.when(i >= 2)
        def _():
            wait_out(j - 2, so)

        transpose(si, so)
        pltpu.async_copy(tout_v.at[so], out_hbm.at[pl.ds(j * 128, 128)],
                         fsem.at[so])
        return carry

    lax.fori_loop(0, PER, blk_body, 0)
    wait_out(j0 + PER - 2, lax.rem(PER - 2, 2))
    wait_out(j0 + PER - 1, lax.rem(PER - 1, 2))

    @pl.when(wid < 4)
    def _():
        j = NFULL + wid
        pltpu.sync_copy(tblt_hbm.at[:, pl.ds(j * 128, 128)], tin_v.at[0])
        transpose(0, 0)
        pltpu.sync_copy(tout_v.at[0], out_hbm.at[pl.ds(j * 128, 128)])

    @pl.when(wid == 4)
    def _():
        pltpu.sync_copy(tail_hbm, tout_v.at[0])
        pltpu.sync_copy(tout_v.at[0], out_hbm.at[pl.ds(VOCAB - 128, 128)])


def _format_table(tblt, tail):
    mesh = plsc.VectorSubcoreMesh(core_axis_name="c", subcore_axis_name="s")
    k = pl.kernel(
        _fmt_body,
        out_type=jax.ShapeDtypeStruct((VP, 128), jnp.float32),
        mesh=mesh,
        scratch_types=[
            pltpu.VMEM((3, D, 128), jnp.float32),
            pltpu.VMEM((2, 128, 128), jnp.float32),
            pltpu.SemaphoreType.DMA((3,)),
            pltpu.SemaphoreType.DMA((2,)),
        ],
        compiler_params=pltpu.CompilerParams(needs_layout_passes=False),
    )
    return k(tblt, tail)


def _pos_diag_order():
    i = np.arange(16)
    d0 = np.arange(16)
    qd = np.arange(4)
    dmat = (d0[:, None] + i[None, :]) % 16
    full = qd[:, None, None] * 16 + dmat[None]
    return jnp.asarray(full.reshape(-1), dtype=jnp.int32)


def kernel(embedding_idx, token_table, pos_table):
    idx_t = embedding_idx.astype(jnp.int32).T
    tail = jnp.pad(token_table[VOCAB - 128:], ((0, 0), (0, 64)))
    tbl2 = _format_table(token_table.T, tail)
    posb = jnp.take(pos_table, _pos_diag_order(), axis=1)
    mesh = plsc.VectorSubcoreMesh(core_axis_name="c", subcore_axis_name="s")
    k = pl.kernel(
        _emb_body,
        out_type=jax.ShapeDtypeStruct((L, D, B), jnp.float32),
        mesh=mesh,
        scratch_types=[
            pltpu.VMEM((L, BT), jnp.int32),
            pltpu.VMEM((NG, D * 16), jnp.float32),
            pltpu.VMEM((NG, BT, 128), jnp.float32),
            pltpu.VMEM((NSO, D, BT), jnp.float32),
            pltpu.SemaphoreType.DMA((NG,)),
            pltpu.SemaphoreType.DMA((NG,)),
            pltpu.SemaphoreType.DMA((NSO,)),
        ],
        compiler_params=pltpu.CompilerParams(needs_layout_passes=False),
    )
    out_t = k(idx_t, tbl2, posb)
    return out_t.transpose(2, 0, 1)


# speedup vs baseline: 2.8147x; 1.1989x over previous
import jax
import jax.numpy as jnp
import numpy as np
from jax import lax
from jax.experimental import pallas as pl
from jax.experimental.pallas import tpu as pltpu
from jax.experimental.pallas import tpu_sc as plsc

B, L, D = 4096, 200, 64
NC, NS = 2, 16
NW = NC * NS
BT = B // NW
NG = 3
NSO = 2
AHEAD = 2


def _emb_body(idx_hbm, tbl_hbm, posb_hbm, out_hbm, idx_v, posl_v, ridx_v,
              rows_v, stage_v, gsem, psem, osem):
    wid = lax.axis_index("s") * NC + lax.axis_index("c")
    b0 = wid * BT
    pltpu.sync_copy(idx_hbm.at[:, pl.ds(b0, BT)], idx_v)

    iota = lax.iota(jnp.int32, 16)
    items = [iota + 16 * g for g in range(BT // 16)]

    def fire(l, sg):
        for g in range(BT // 16):
            sl = pl.ds(16 * g, 16)
            ridx_v[sg, sl] = lax.shift_right_logical(idx_v[l, sl], 1)
        pltpu.async_copy(tbl_hbm.at[ridx_v.at[sg]], rows_v.at[sg],
                         gsem.at[sg])
        pltpu.async_copy(posb_hbm.at[l], posl_v.at[sg], psem.at[sg])

    def wait_gather(l, sg):
        pltpu.make_async_copy(tbl_hbm.at[ridx_v.at[sg]], rows_v.at[sg],
                              gsem.at[sg]).wait()
        pltpu.make_async_copy(posb_hbm.at[l], posl_v.at[sg],
                              psem.at[sg]).wait()

    def wait_out(l, so):
        pltpu.make_async_copy(stage_v.at[so], out_hbm.at[l, :, pl.ds(b0, BT)],
                              osem.at[so]).wait()

    for l in range(AHEAD):
        fire(l, l % NG)

    def pos_body(l, carry):
        ln = l + AHEAD

        @pl.when(ln < L)
        def _():
            fire(ln, lax.rem(ln, NG))

        sg = lax.rem(l, NG)
        so = lax.rem(l, NSO)
        wait_gather(l, sg)

        @pl.when(l >= NSO)
        def _():
            wait_out(l - NSO, so)

        par = [(idx_v[l, pl.ds(16 * g, 16)] & 1) * 64
               for g in range(BT // 16)]
        rows2 = rows_v.at[sg]

        def d0_body(d0):
            rot = (iota + d0) & 15
            for qd in range(D // 16):
                rotq = rot + 16 * qd
                pos_vec = posl_v[sg, pl.ds(256 * qd + d0 * 16, 16)]
                for g in range(BT // 16):
                    vals = plsc.load_gather(rows2, [items[g], par[g] + rotq])
                    plsc.store_scatter(stage_v.at[so], [rotq, items[g]],
                                       vals + pos_vec)

        plsc.parallel_loop(0, 16, 1, unroll=2)(d0_body)
        pltpu.async_copy(stage_v.at[so], out_hbm.at[l, :, pl.ds(b0, BT)],
                         osem.at[so])
        return carry

    lax.fori_loop(0, L, pos_body, 0)

    for k in range(NSO):
        l = L - NSO + k
        wait_out(l, l % NSO)


VOCAB = 1000000
VP = VOCAB // 2
NFULL = 7808
PER = NFULL // NW


def _fmt_body(tblt_hbm, tail_hbm, out_hbm, tin_v, tout_v, isem, fsem):
    wid = lax.axis_index("s") * NC + lax.axis_index("c")
    iota = lax.iota(jnp.int32, 16)
    items = [iota + 16 * g for g in range(8)]

    def fire_in(j, s):
        pltpu.async_copy(tblt_hbm.at[:, pl.ds(j * 128, 128)], tin_v.at[s],
                         isem.at[s])

    def wait_in(j, s):
        pltpu.make_async_copy(tblt_hbm.at[:, pl.ds(j * 128, 128)],
                              tin_v.at[s], isem.at[s]).wait()

    def wait_out(j, s):
        pltpu.make_async_copy(tout_v.at[s], out_hbm.at[pl.ds(j * 64, 64)],
                              fsem.at[s]).wait()

    vrows = [(iota >> 1) + 8 * g for g in range(8)]
    parcol = (iota & 1) * 64

    def transpose(si, so):
        def d0_body(d0):
            rot = (iota + d0) & 15
            for qd in range(D // 16):
                rotq = rot + 16 * qd
                colv = parcol + rotq
                for g in range(8):
                    vals = plsc.load_gather(tin_v.at[si], [rotq, items[g]])
                    plsc.store_scatter(tout_v.at[so], [vrows[g], colv], vals)

        plsc.parallel_loop(0, 16, 1, unroll=2)(d0_body)

    j0 = wid * PER
    for i in range(2):
        fire_in(j0 + i, i % 3)

    def blk_body(i, carry):
        j = j0 + i

        @pl.when(i + 2 < PER)
        def _():
            fire_in(j + 2, lax.rem(i + 2, 3))

        si = lax.rem(i, 3)
        so = lax.rem(i, 2)
        wait_in(j, si)

        @pl.when(i >= 2)
        def _():
            wait_out(j - 2, so)

        transpose(si, so)
        pltpu.async_copy(tout_v.at[so], out_hbm.at[pl.ds(j * 64, 64)],
                         fsem.at[so])
        return carry

    lax.fori_loop(0, PER, blk_body, 0)
    wait_out(j0 + PER - 2, lax.rem(PER - 2, 2))
    wait_out(j0 + PER - 1, lax.rem(PER - 1, 2))

    @pl.when(wid < 4)
    def _():
        j = NFULL + wid
        pltpu.sync_copy(tblt_hbm.at[:, pl.ds(j * 128, 128)], tin_v.at[0])
        transpose(0, 0)
        pltpu.sync_copy(tout_v.at[0], out_hbm.at[pl.ds(j * 64, 64)])

    @pl.when(wid == 4)
    def _():
        pltpu.sync_copy(tail_hbm, tout_v.at[0])
        pltpu.sync_copy(tout_v.at[0], out_hbm.at[pl.ds(VP - 64, 64)])


def _format_table(tblt, tail):
    mesh = plsc.VectorSubcoreMesh(core_axis_name="c", subcore_axis_name="s")
    k = pl.kernel(
        _fmt_body,
        out_type=jax.ShapeDtypeStruct((VP, 128), jnp.float32),
        mesh=mesh,
        scratch_types=[
            pltpu.VMEM((3, D, 128), jnp.float32),
            pltpu.VMEM((2, 64, 128), jnp.float32),
            pltpu.SemaphoreType.DMA((3,)),
            pltpu.SemaphoreType.DMA((2,)),
        ],
        compiler_params=pltpu.CompilerParams(needs_layout_passes=False),
    )
    return k(tblt, tail)


def _pos_diag_order():
    i = np.arange(16)
    d0 = np.arange(16)
    qd = np.arange(4)
    dmat = (d0[:, None] + i[None, :]) % 16
    full = qd[:, None, None] * 16 + dmat[None]
    return jnp.asarray(full.reshape(-1), dtype=jnp.int32)


def kernel(embedding_idx, token_table, pos_table):
    idx_t = embedding_idx.astype(jnp.int32).T
    tail = token_table[VOCAB - 128:].reshape(64, 128)
    tbl2 = _format_table(token_table.T, tail)
    posb = jnp.take(pos_table, _pos_diag_order(), axis=1)
    mesh = plsc.VectorSubcoreMesh(core_axis_name="c", subcore_axis_name="s")
    k = pl.kernel(
        _emb_body,
        out_type=jax.ShapeDtypeStruct((L, D, B), jnp.float32),
        mesh=mesh,
        scratch_types=[
            pltpu.VMEM((L, BT), jnp.int32),
            pltpu.VMEM((NG, D * 16), jnp.float32),
            pltpu.VMEM((NG, BT), jnp.int32),
            pltpu.VMEM((NG, BT, 128), jnp.float32),
            pltpu.VMEM((NSO, D, BT), jnp.float32),
            pltpu.SemaphoreType.DMA((NG,)),
            pltpu.SemaphoreType.DMA((NG,)),
            pltpu.SemaphoreType.DMA((NSO,)),
        ],
        compiler_params=pltpu.CompilerParams(needs_layout_passes=False),
    )
    out_t = k(idx_t, tbl2, posb)
    return out_t.transpose(2, 0, 1)


# speedup vs baseline: 2.8552x; 1.0144x over previous
import jax
import jax.numpy as jnp
import numpy as np
from jax import lax
from jax.experimental import pallas as pl
from jax.experimental.pallas import tpu as pltpu
from jax.experimental.pallas import tpu_sc as plsc

B, L, D = 4096, 200, 64
NC, NS = 2, 16
NW = NC * NS
BT = B // NW
NG = 4
NSO = 2
AHEAD = 3


def _emb_body(idx_hbm, tbl_hbm, posb_hbm, out_hbm, idx_v, posl_v, ridx_v,
              rows_v, stage_v, gsem, psem, osem):
    wid = lax.axis_index("s") * NC + lax.axis_index("c")
    b0 = wid * BT
    pltpu.sync_copy(idx_hbm.at[:, pl.ds(b0, BT)], idx_v)

    iota = lax.iota(jnp.int32, 16)
    items = [iota + 16 * g for g in range(BT // 16)]

    def fire(l, sg):
        for g in range(BT // 16):
            sl = pl.ds(16 * g, 16)
            ridx_v[sg, sl] = lax.shift_right_logical(idx_v[l, sl], 1)
        pltpu.async_copy(tbl_hbm.at[ridx_v.at[sg]], rows_v.at[sg],
                         gsem.at[sg])
        pltpu.async_copy(posb_hbm.at[l], posl_v.at[sg], psem.at[sg])

    def wait_gather(l, sg):
        pltpu.make_async_copy(tbl_hbm.at[ridx_v.at[sg]], rows_v.at[sg],
                              gsem.at[sg]).wait()
        pltpu.make_async_copy(posb_hbm.at[l], posl_v.at[sg],
                              psem.at[sg]).wait()

    def wait_out(l, so):
        pltpu.make_async_copy(stage_v.at[so], out_hbm.at[l, :, pl.ds(b0, BT)],
                              osem.at[so]).wait()

    for l in range(AHEAD):
        fire(l, l % NG)

    def pos_body(l, carry):
        ln = l + AHEAD

        @pl.when(ln < L)
        def _():
            fire(ln, lax.rem(ln, NG))

        sg = lax.rem(l, NG)
        so = lax.rem(l, NSO)
        wait_gather(l, sg)

        @pl.when(l >= NSO)
        def _():
            wait_out(l - NSO, so)

        par = [(idx_v[l, pl.ds(16 * g, 16)] & 1) * 64
               for g in range(BT // 16)]
        rows2 = rows_v.at[sg]

        def d0_body(d0):
            rot = (iota + d0) & 15
            for qd in range(D // 16):
                rotq = rot + 16 * qd
                pos_vec = posl_v[sg, pl.ds(256 * qd + d0 * 16, 16)]
                for g in range(BT // 16):
                    vals = plsc.load_gather(rows2, [items[g], par[g] + rotq])
                    plsc.store_scatter(stage_v.at[so], [rotq, items[g]],
                                       vals + pos_vec)

        plsc.parallel_loop(0, 16, 1, unroll=2)(d0_body)
        pltpu.async_copy(stage_v.at[so], out_hbm.at[l, :, pl.ds(b0, BT)],
                         osem.at[so])
        return carry

    lax.fori_loop(0, L, pos_body, 0)

    for k in range(NSO):
        l = L - NSO + k
        wait_out(l, l % NSO)


VOCAB = 1000000
VP = VOCAB // 2
NFULL = 7808
PER = NFULL // NW


def _fmt_body(tblt_hbm, tail_hbm, out_hbm, tin_v, tout_v, isem, fsem):
    wid = lax.axis_index("s") * NC + lax.axis_index("c")
    iota = lax.iota(jnp.int32, 16)
    items = [iota + 16 * g for g in range(8)]

    def fire_in(j, s):
        pltpu.async_copy(tblt_hbm.at[:, pl.ds(j * 128, 128)], tin_v.at[s],
                         isem.at[s])

    def wait_in(j, s):
        pltpu.make_async_copy(tblt_hbm.at[:, pl.ds(j * 128, 128)],
                              tin_v.at[s], isem.at[s]).wait()

    def wait_out(j, s):
        pltpu.make_async_copy(tout_v.at[s], out_hbm.at[pl.ds(j * 64, 64)],
                              fsem.at[s]).wait()

    vrows = [(iota >> 1) + 8 * g for g in range(8)]
    parcol = (iota & 1) * 64

    def transpose(si, so):
        def d0_body(d0):
            rot = (iota + d0) & 15
            for qd in range(D // 16):
                rotq = rot + 16 * qd
                colv = parcol + rotq
                for g in range(8):
                    vals = plsc.load_gather(tin_v.at[si], [rotq, items[g]])
                    plsc.store_scatter(tout_v.at[so], [vrows[g], colv], vals)

        plsc.parallel_loop(0, 16, 1, unroll=2)(d0_body)

    j0 = wid * PER
    for i in range(2):
        fire_in(j0 + i, i % 3)

    def blk_body(i, carry):
        j = j0 + i

        @pl.when(i + 2 < PER)
        def _():
            fire_in(j + 2, lax.rem(i + 2, 3))

        si = lax.rem(i, 3)
        so = lax.rem(i, 2)
        wait_in(j, si)

        @pl.when(i >= 2)
        def _():
            wait_out(j - 2, so)

        transpose(si, so)
        pltpu.async_copy(tout_v.at[so], out_hbm.at[pl.ds(j * 64, 64)],
                         fsem.at[so])
        return carry

    lax.fori_loop(0, PER, blk_body, 0)
    wait_out(j0 + PER - 2, lax.rem(PER - 2, 2))
    wait_out(j0 + PER - 1, lax.rem(PER - 1, 2))

    @pl.when(wid < 4)
    def _():
        j = NFULL + wid
        pltpu.sync_copy(tblt_hbm.at[:, pl.ds(j * 128, 128)], tin_v.at[0])
        transpose(0, 0)
        pltpu.sync_copy(tout_v.at[0], out_hbm.at[pl.ds(j * 64, 64)])

    @pl.when(wid == 4)
    def _():
        pltpu.sync_copy(tail_hbm, tout_v.at[0])
        pltpu.sync_copy(tout_v.at[0], out_hbm.at[pl.ds(VP - 64, 64)])


def _format_table(tblt, tail):
    mesh = plsc.VectorSubcoreMesh(core_axis_name="c", subcore_axis_name="s")
    k = pl.kernel(
        _fmt_body,
        out_type=jax.ShapeDtypeStruct((VP, 128), jnp.float32),
        mesh=mesh,
        scratch_types=[
            pltpu.VMEM((3, D, 128), jnp.float32),
            pltpu.VMEM((2, 64, 128), jnp.float32),
            pltpu.SemaphoreType.DMA((3,)),
            pltpu.SemaphoreType.DMA((2,)),
        ],
        compiler_params=pltpu.CompilerParams(needs_layout_passes=False),
    )
    return k(tblt, tail)


def _pos_diag_order():
    i = np.arange(16)
    d0 = np.arange(16)
    qd = np.arange(4)
    dmat = (d0[:, None] + i[None, :]) % 16
    full = qd[:, None, None] * 16 + dmat[None]
    return jnp.asarray(full.reshape(-1), dtype=jnp.int32)


def kernel(embedding_idx, token_table, pos_table):
    idx_t = embedding_idx.astype(jnp.int32).T
    tail = token_table[VOCAB - 128:].reshape(64, 128)
    tbl2 = _format_table(token_table.T, tail)
    posb = jnp.take(pos_table, _pos_diag_order(), axis=1)
    mesh = plsc.VectorSubcoreMesh(core_axis_name="c", subcore_axis_name="s")
    k = pl.kernel(
        _emb_body,
        out_type=jax.ShapeDtypeStruct((L, D, B), jnp.float32),
        mesh=mesh,
        scratch_types=[
            pltpu.VMEM((L, BT), jnp.int32),
            pltpu.VMEM((NG, D * 16), jnp.float32),
            pltpu.VMEM((NG, BT), jnp.int32),
            pltpu.VMEM((NG, BT, 128), jnp.float32),
            pltpu.VMEM((NSO, D, BT), jnp.float32),
            pltpu.SemaphoreType.DMA((NG,)),
            pltpu.SemaphoreType.DMA((NG,)),
            pltpu.SemaphoreType.DMA((NSO,)),
        ],
        compiler_params=pltpu.CompilerParams(needs_layout_passes=False),
    )
    out_t = k(idx_t, tbl2, posb)
    return out_t.transpose(2, 0, 1)


# speedup vs baseline: 2.9574x; 1.0358x over previous
import jax
import jax.numpy as jnp
import numpy as np
from jax import lax
from jax.experimental import pallas as pl
from jax.experimental.pallas import tpu as pltpu
from jax.experimental.pallas import tpu_sc as plsc

B, L, D = 4096, 200, 64
NC, NS = 2, 16
NW = NC * NS
BT = B // NW
NG = 4
NSO = 3
AHEAD = 3


def _emb_body(idx_hbm, tbl_hbm, posb_hbm, out_hbm, idx_v, posl_v, ridx_v,
              rows_v, stage_v, gsem, psem, osem):
    wid = lax.axis_index("s") * NC + lax.axis_index("c")
    b0 = wid * BT
    pltpu.sync_copy(idx_hbm.at[:, pl.ds(b0, BT)], idx_v)

    iota = lax.iota(jnp.int32, 16)
    items = [iota + 16 * g for g in range(BT // 16)]

    def fire(l, sg):
        for g in range(BT // 16):
            sl = pl.ds(16 * g, 16)
            ridx_v[sg, sl] = lax.shift_right_logical(idx_v[l, sl], 1)
        pltpu.async_copy(tbl_hbm.at[ridx_v.at[sg]], rows_v.at[sg],
                         gsem.at[sg])
        pltpu.async_copy(posb_hbm.at[l], posl_v.at[sg], psem.at[sg])

    def wait_gather(l, sg):
        pltpu.make_async_copy(tbl_hbm.at[ridx_v.at[sg]], rows_v.at[sg],
                              gsem.at[sg]).wait()
        pltpu.make_async_copy(posb_hbm.at[l], posl_v.at[sg],
                              psem.at[sg]).wait()

    def wait_out(l, so):
        pltpu.make_async_copy(stage_v.at[so], out_hbm.at[l, :, pl.ds(b0, BT)],
                              osem.at[so]).wait()

    for l in range(AHEAD):
        fire(l, l % NG)

    def pos_body(l, carry):
        ln = l + AHEAD

        @pl.when(ln < L)
        def _():
            fire(ln, lax.rem(ln, NG))

        sg = lax.rem(l, NG)
        so = lax.rem(l, NSO)
        wait_gather(l, sg)

        @pl.when(l >= NSO)
        def _():
            wait_out(l - NSO, so)

        par = [(idx_v[l, pl.ds(16 * g, 16)] & 1) * 64
               for g in range(BT // 16)]
        rows2 = rows_v.at[sg]

        def d0_body(d0):
            rot = (iota + d0) & 15
            for qd in range(D // 16):
                rotq = rot + 16 * qd
                pos_vec = posl_v[sg, pl.ds(256 * qd + d0 * 16, 16)]
                for g in range(BT // 16):
                    vals = plsc.load_gather(rows2, [items[g], par[g] + rotq])
                    plsc.store_scatter(stage_v.at[so], [rotq, items[g]],
                                       vals + pos_vec)

        plsc.parallel_loop(0, 16, 1, unroll=2)(d0_body)
        pltpu.async_copy(stage_v.at[so], out_hbm.at[l, :, pl.ds(b0, BT)],
                         osem.at[so])
        return carry

    lax.fori_loop(0, L, pos_body, 0)

    for k in range(NSO):
        l = L - NSO + k
        wait_out(l, l % NSO)


VOCAB = 1000000
VP = VOCAB // 2
NFULL = 7808
PER = NFULL // NW


def _fmt_body(tblt_hbm, tail_hbm, out_hbm, tin_v, tout_v, isem, fsem):
    wid = lax.axis_index("s") * NC + lax.axis_index("c")
    iota = lax.iota(jnp.int32, 16)
    items = [iota + 16 * g for g in range(8)]

    def fire_in(j, s):
        pltpu.async_copy(tblt_hbm.at[:, pl.ds(j * 128, 128)], tin_v.at[s],
                         isem.at[s])

    def wait_in(j, s):
        pltpu.make_async_copy(tblt_hbm.at[:, pl.ds(j * 128, 128)],
                              tin_v.at[s], isem.at[s]).wait()

    def wait_out(j, s):
        pltpu.make_async_copy(tout_v.at[s], out_hbm.at[pl.ds(j * 64, 64)],
                              fsem.at[s]).wait()

    vrows = [(iota >> 1) + 8 * g for g in range(8)]
    parcol = (iota & 1) * 64

    def transpose(si, so):
        def d0_body(d0):
            rot = (iota + d0) & 15
            for qd in range(D // 16):
                rotq = rot + 16 * qd
                colv = parcol + rotq
                for g in range(8):
                    vals = plsc.load_gather(tin_v.at[si], [rotq, items[g]])
                    plsc.store_scatter(tout_v.at[so], [vrows[g], colv], vals)

        plsc.parallel_loop(0, 16, 1, unroll=2)(d0_body)

    j0 = wid * PER
    for i in range(3):
        fire_in(j0 + i, i % 4)

    def blk_body(i, carry):
        j = j0 + i

        @pl.when(i + 3 < PER)
        def _():
            fire_in(j + 3, lax.rem(i + 3, 4))

        si = lax.rem(i, 4)
        so = lax.rem(i, 3)
        wait_in(j, si)

        @pl.when(i >= 3)
        def _():
            wait_out(j - 3, so)

        transpose(si, so)
        pltpu.async_copy(tout_v.at[so], out_hbm.at[pl.ds(j * 64, 64)],
                         fsem.at[so])
        return carry

    lax.fori_loop(0, PER, blk_body, 0)
    for t in range(3):
        wait_out(j0 + PER - 3 + t, lax.rem(PER - 3 + t, 3))

    @pl.when(wid < 4)
    def _():
        j = NFULL + wid
        pltpu.sync_copy(tblt_hbm.at[:, pl.ds(j * 128, 128)], tin_v.at[0])
        transpose(0, 0)
        pltpu.sync_copy(tout_v.at[0], out_hbm.at[pl.ds(j * 64, 64)])

    @pl.when(wid == 4)
    def _():
        pltpu.sync_copy(tail_hbm, tout_v.at[0])
        pltpu.sync_copy(tout_v.at[0], out_hbm.at[pl.ds(VP - 64, 64)])


def _format_table(tblt, tail):
    mesh = plsc.VectorSubcoreMesh(core_axis_name="c", subcore_axis_name="s")
    k = pl.kernel(
        _fmt_body,
        out_type=jax.ShapeDtypeStruct((VP, 128), jnp.float32),
        mesh=mesh,
        scratch_types=[
            pltpu.VMEM((4, D, 128), jnp.float32),
            pltpu.VMEM((3, 64, 128), jnp.float32),
            pltpu.SemaphoreType.DMA((4,)),
            pltpu.SemaphoreType.DMA((3,)),
        ],
        compiler_params=pltpu.CompilerParams(needs_layout_passes=False),
    )
    return k(tblt, tail)


def _pos_diag_order():
    i = np.arange(16)
    d0 = np.arange(16)
    qd = np.arange(4)
    dmat = (d0[:, None] + i[None, :]) % 16
    full = qd[:, None, None] * 16 + dmat[None]
    return jnp.asarray(full.reshape(-1), dtype=jnp.int32)


def kernel(embedding_idx, token_table, pos_table):
    idx_t = embedding_idx.astype(jnp.int32).T
    tail = token_table[VOCAB - 128:].reshape(64, 128)
    tbl2 = _format_table(token_table.T, tail)
    posb = jnp.take(pos_table, _pos_diag_order(), axis=1)
    mesh = plsc.VectorSubcoreMesh(core_axis_name="c", subcore_axis_name="s")
    k = pl.kernel(
        _emb_body,
        out_type=jax.ShapeDtypeStruct((L, D, B), jnp.float32),
        mesh=mesh,
        scratch_types=[
            pltpu.VMEM((L, BT), jnp.int32),
            pltpu.VMEM((NG, D * 16), jnp.float32),
            pltpu.VMEM((NG, BT), jnp.int32),
            pltpu.VMEM((NG, BT, 128), jnp.float32),
            pltpu.VMEM((NSO, D, BT), jnp.float32),
            pltpu.SemaphoreType.DMA((NG,)),
            pltpu.SemaphoreType.DMA((NG,)),
            pltpu.SemaphoreType.DMA((NSO,)),
        ],
        compiler_params=pltpu.CompilerParams(needs_layout_passes=False),
    )
    out_t = k(idx_t, tbl2, posb)
    return out_t.transpose(2, 0, 1)
